# baseline (speedup 1.0000x reference)
import jax
import jax.numpy as jnp
from jax import lax
from jax.experimental import pallas as pl
from jax.experimental.pallas import tpu as pltpu
from jax.experimental.pallas import tpu_sc as plsc

VOCAB = 100000
MAX_LEN = 200
HIDDEN = 128
BATCH = 1024
N_TOK = BATCH * MAX_LEN
NW = 32
TOK_PER_W = N_TOK // NW
CHUNK = 128
NCHUNK = TOK_PER_W // CHUNK
NCOMBO = 2 * MAX_LEN
TRIPLES = (NCHUNK - 2) // 3
NJ = HIDDEN // 16


def _sc_body(tok_table, combo, tok_idx, cmb_idx, gamma, beta, out,
             tok_idx_v, cmb_idx_v, combo_v, buf, gv, bv,
             sbuf, s2buf, mbuf, rbuf,
             tsem, asem, osem):
  wid = lax.axis_index("s") * 2 + lax.axis_index("c")
  w_base = wid * TOK_PER_W

  pltpu.sync_copy(gamma, gv)
  pltpu.sync_copy(beta, bv)
  pltpu.sync_copy(combo, combo_v)
  pltpu.sync_copy(tok_idx.at[wid], tok_idx_v)
  pltpu.sync_copy(cmb_idx.at[wid], cmb_idx_v)
  gvs = [gv[pl.ds(16 * j, 16)] for j in range(NJ)]
  bvs = [bv[pl.ds(16 * j, 16)] for j in range(NJ)]

  lanes = lax.iota(jnp.int32, 16)

  def issue_tok(g, s):
    pltpu.async_copy(tok_table.at[tok_idx_v.at[g]], buf.at[s], tsem.at[s])

  def wait_tok(g, s):
    pltpu.make_async_copy(tok_table.at[tok_idx_v.at[g]], buf.at[s],
                          tsem.at[s]).wait()

  def issue_add(g, s):
    pltpu.async_copy(combo_v.at[cmb_idx_v.at[g]], buf.at[s], asem.at[s],
                     add=True)

  def wait_add(g, s):
    pltpu.make_async_copy(combo_v.at[cmb_idx_v.at[g]], buf.at[s],
                          asem.at[s]).wait()

  def out_copy(g, s):
    base = w_base + g * CHUNK
    return pltpu.make_async_copy(buf.at[s], out.at[pl.ds(base, CHUNK)],
                                 osem.at[s])

  def compute(s):
    def group(grp, carry):
      @plsc.parallel_loop(0, 16, 1, unroll=4)
      def _(ti):
        t = grp * 16 + ti
        y = [buf[s, t, pl.ds(16 * j, 16)] for j in range(NJ)]
        tot = ((y[0] + y[1]) + (y[2] + y[3])) + ((y[4] + y[5]) + (y[6] + y[7]))
        q = [yj * yj for yj in y]
        sq = ((q[0] + q[1]) + (q[2] + q[3])) + ((q[4] + q[5]) + (q[6] + q[7]))
        sbuf[pl.ds(ti * 17, 16)] = tot
        s2buf[pl.ds(ti * 17, 16)] = sq

      zero = jnp.zeros((16,), jnp.float32)

      @plsc.parallel_loop(0, 16, 1, unroll=4,
                          carry=(lanes * 17, zero, zero))
      def red(k, c):
        ck, acc, acc2 = c
        acc = acc + plsc.load_gather(sbuf, (ck,))
        acc2 = acc2 + plsc.load_gather(s2buf, (ck,))
        return ck + 1, acc, acc2

      _, acc, acc2 = red
      mean = acc * (1.0 / HIDDEN)
      var = acc2 * (1.0 / HIDDEN) - mean * mean
      a = var + 1e-5
      yi = jnp.int32(0x5F3759DF) - (plsc.bitcast(a, jnp.int32) >> 1)
      r = plsc.bitcast(yi, jnp.float32)
      h = a * 0.5
      for _ in range(3):
        r = r * (1.5 - h * r * r)
      mbuf[:] = mean * r
      rbuf[:] = r

      @plsc.parallel_loop(0, 16, 1, unroll=4)
      def _(ti):
        t = grp * 16 + ti
        tsplat = jnp.full((16,), ti, jnp.int32)
        mr = plsc.load_gather(mbuf, (tsplat,))
        rs = plsc.load_gather(rbuf, (tsplat,))
        for j in range(NJ):
          yj = buf[s, t, pl.ds(16 * j, 16)]
          buf[s, t, pl.ds(16 * j, 16)] = (yj * rs - mr) * gvs[j] + bvs[j]
      return carry

    lax.fori_loop(0, CHUNK // 16, group, 0)

  def step(g, s, first):
    s1 = (s + 1) % 3
    s2 = (s + 2) % 3
    wait_tok(g + 1, s1)
    issue_add(g + 1, s1)
    wait_add(g, s)
    compute(s)
    out_copy(g, s).start()
    if not first:
      out_copy(g - 1, s2).wait()
    issue_tok(g + 2, s2)

  issue_tok(0, 0)
  issue_tok(1, 1)
  wait_tok(0, 0)
  issue_add(0, 0)

  def triple(p, carry, first):
    g = 3 * p
    step(g, 0, first)
    step(g + 1, 1, False)
    step(g + 2, 2, False)
    return carry

  triple(0, 0, True)
  lax.fori_loop(1, TRIPLES, lambda p, c: triple(p, c, False), 0)

  g = NCHUNK - 2
  wait_tok(g + 1, 1)
  issue_add(g + 1, 1)
  wait_add(g, 0)
  compute(0)
  out_copy(g, 0).start()

  g = NCHUNK - 1
  wait_add(g, 1)
  compute(1)
  out_copy(g, 1).start()

  out_copy(NCHUNK - 3, 2).wait()
  out_copy(NCHUNK - 2, 0).wait()
  out_copy(NCHUNK - 1, 1).wait()


_sc_call = pl.kernel(
    _sc_body,
    out_type=jax.ShapeDtypeStruct((N_TOK, HIDDEN), jnp.float32),
    mesh=plsc.VectorSubcoreMesh(core_axis_name="c", subcore_axis_name="s"),
    compiler_params=pltpu.CompilerParams(needs_layout_passes=False),
    scratch_types=[
        pltpu.VMEM((NCHUNK, CHUNK), jnp.int32),
        pltpu.VMEM((NCHUNK, CHUNK), jnp.int32),
        pltpu.VMEM_SHARED((NCOMBO, HIDDEN), jnp.float32),
        pltpu.VMEM((3, CHUNK, HIDDEN), jnp.float32),
        pltpu.VMEM((HIDDEN,), jnp.float32),
        pltpu.VMEM((HIDDEN,), jnp.float32),
        pltpu.VMEM((16 * 17,), jnp.float32),
        pltpu.VMEM((16 * 17,), jnp.float32),
        pltpu.VMEM((16,), jnp.float32),
        pltpu.VMEM((16,), jnp.float32),
        pltpu.SemaphoreType.DMA((3,)),
        pltpu.SemaphoreType.DMA((3,)),
        pltpu.SemaphoreType.DMA((3,)),
    ],
)


def kernel(input_token, segment_ids, token_table, type_table, pos_table,
           gamma, beta):
  tok_idx = input_token.reshape(NW, NCHUNK, CHUNK)
  cmb_idx = (2 * jnp.arange(MAX_LEN, dtype=jnp.int32)[None, :]
             + segment_ids).reshape(NW, NCHUNK, CHUNK)
  combo = (pos_table[:, None, :] + type_table[None, :, :]).reshape(
      NCOMBO, HIDDEN)
  out = _sc_call(token_table, combo, tok_idx, cmb_idx, gamma, beta)
  return out.reshape(BATCH, MAX_LEN, HIDDEN)


# speedup vs baseline: 1.1062x; 1.1062x over previous
import jax
import jax.numpy as jnp
from jax import lax
from jax.experimental import pallas as pl
from jax.experimental.pallas import tpu as pltpu
from jax.experimental.pallas import tpu_sc as plsc

VOCAB = 100000
MAX_LEN = 200
HIDDEN = 128
BATCH = 1024
N_TOK = BATCH * MAX_LEN
NW = 32
TOK_PER_W = N_TOK // NW
CHUNK = 128
NCHUNK = TOK_PER_W // CHUNK
POSX = MAX_LEN + CHUNK
TRIPLES = (NCHUNK - 2) // 3
NJ = HIDDEN // 16


def _sc_body(tok_table, posx, tok_idx, seg, gamma, beta, dvec, out,
             tok_idx_v, seg_v, posx_v, buf, gv, bv, dv,
             sbuf, s2buf, mbuf, rbuf,
             tsem, osem):
  wid = lax.axis_index("s") * 2 + lax.axis_index("c")
  w_base = wid * TOK_PER_W

  pltpu.sync_copy(gamma, gv)
  pltpu.sync_copy(beta, bv)
  pltpu.sync_copy(dvec, dv)
  pltpu.sync_copy(posx, posx_v)
  pltpu.sync_copy(tok_idx.at[wid], tok_idx_v)
  pltpu.sync_copy(seg.at[wid], seg_v)
  gvs = [gv[pl.ds(16 * j, 16)] for j in range(NJ)]
  bvs = [bv[pl.ds(16 * j, 16)] for j in range(NJ)]
  dvs = [dv[pl.ds(16 * j, 16)] for j in range(NJ)]

  lanes = lax.iota(jnp.int32, 16)
  zeros16i = jnp.zeros((16,), jnp.int32)

  def issue_tok(g, s):
    pltpu.async_copy(tok_table.at[tok_idx_v.at[g]], buf.at[s], tsem.at[s])

  def wait_tok(g, s):
    pltpu.make_async_copy(tok_table.at[tok_idx_v.at[g]], buf.at[s],
                          tsem.at[s]).wait()

  def out_copy(g, s):
    base = w_base + g * CHUNK
    return pltpu.make_async_copy(buf.at[s], out.at[pl.ds(base, CHUNK)],
                                 osem.at[s])

  def compute(g, s, lbase):
    def group(grp, carry):
      @plsc.parallel_loop(0, 16, 1, unroll=8)
      def _(ti):
        t = grp * 16 + ti
        sseg = plsc.load_gather(seg_v, (zeros16i + (g * CHUNK + t),))
        prow = lbase + t
        y = [buf[s, t, pl.ds(16 * j, 16)] + posx_v[prow, pl.ds(16 * j, 16)]
             + sseg * dvs[j] for j in range(NJ)]
        for j in range(NJ):
          buf[s, t, pl.ds(16 * j, 16)] = y[j]
        tot = ((y[0] + y[1]) + (y[2] + y[3])) + ((y[4] + y[5]) + (y[6] + y[7]))
        q = [yj * yj for yj in y]
        sq = ((q[0] + q[1]) + (q[2] + q[3])) + ((q[4] + q[5]) + (q[6] + q[7]))
        sbuf[pl.ds(ti * 17, 16)] = tot
        s2buf[pl.ds(ti * 17, 16)] = sq

      zero = jnp.zeros((16,), jnp.float32)

      @plsc.parallel_loop(0, 16, 1, unroll=8,
                          carry=(lanes * 17, zero, zero))
      def red(k, c):
        ck, acc, acc2 = c
        acc = acc + plsc.load_gather(sbuf, (ck,))
        acc2 = acc2 + plsc.load_gather(s2buf, (ck,))
        return ck + 1, acc, acc2

      _, acc, acc2 = red
      mean = acc * (1.0 / HIDDEN)
      var = acc2 * (1.0 / HIDDEN) - mean * mean
      a = var + 1e-5
      yi = jnp.int32(0x5F3759DF) - (plsc.bitcast(a, jnp.int32) >> 1)
      r = plsc.bitcast(yi, jnp.float32)
      h = a * 0.5
      for _ in range(3):
        r = r * (1.5 - h * r * r)
      mbuf[:] = mean * r
      rbuf[:] = r

      @plsc.parallel_loop(0, 16, 1, unroll=8)
      def _(ti):
        t = grp * 16 + ti
        tsplat = jnp.full((16,), ti, jnp.int32)
        mr = plsc.load_gather(mbuf, (tsplat,))
        rs = plsc.load_gather(rbuf, (tsplat,))
        for j in range(NJ):
          yj = buf[s, t, pl.ds(16 * j, 16)]
          buf[s, t, pl.ds(16 * j, 16)] = (yj * rs - mr) * gvs[j] + bvs[j]
      return carry

    lax.fori_loop(0, CHUNK // 16, group, 0)

  def wrap(x):
    return jnp.where(x >= MAX_LEN, x - MAX_LEN, x)

  def step(g, s, lbase, first):
    s2 = (s + 2) % 3
    wait_tok(g, s)
    compute(g, s, lbase)
    out_copy(g, s).start()
    if not first:
      out_copy(g - 1, s2).wait()
    issue_tok(g + 2, s2)

  issue_tok(0, 0)
  issue_tok(1, 1)

  def triple(p, lbase, first):
    g = 3 * p
    step(g, 0, lbase, first)
    lbase = wrap(lbase + CHUNK)
    step(g + 1, 1, lbase, False)
    lbase = wrap(lbase + CHUNK)
    step(g + 2, 2, lbase, False)
    return wrap(lbase + CHUNK)

  lbase = triple(0, 0, True)
  lbase = lax.fori_loop(1, TRIPLES, lambda p, lb: triple(p, lb, False), lbase)

  g = NCHUNK - 2
  wait_tok(g, 0)
  compute(g, 0, lbase)
  out_copy(g, 0).start()

  g = NCHUNK - 1
  lbase = wrap(lbase + CHUNK)
  wait_tok(g, 1)
  compute(g, 1, lbase)
  out_copy(g, 1).start()

  out_copy(NCHUNK - 3, 2).wait()
  out_copy(NCHUNK - 2, 0).wait()
  out_copy(NCHUNK - 1, 1).wait()


_sc_call = pl.kernel(
    _sc_body,
    out_type=jax.ShapeDtypeStruct((N_TOK, HIDDEN), jnp.float32),
    mesh=plsc.VectorSubcoreMesh(core_axis_name="c", subcore_axis_name="s"),
    compiler_params=pltpu.CompilerParams(needs_layout_passes=False),
    scratch_types=[
        pltpu.VMEM((NCHUNK, CHUNK), jnp.int32),
        pltpu.VMEM((NCHUNK * CHUNK,), jnp.float32),
        pltpu.VMEM((POSX, HIDDEN), jnp.float32),
        pltpu.VMEM((3, CHUNK, HIDDEN), jnp.float32),
        pltpu.VMEM((HIDDEN,), jnp.float32),
        pltpu.VMEM((HIDDEN,), jnp.float32),
        pltpu.VMEM((HIDDEN,), jnp.float32),
        pltpu.VMEM((16 * 17,), jnp.float32),
        pltpu.VMEM((16 * 17,), jnp.float32),
        pltpu.VMEM((16,), jnp.float32),
        pltpu.VMEM((16,), jnp.float32),
        pltpu.SemaphoreType.DMA((3,)),
        pltpu.SemaphoreType.DMA((3,)),
    ],
)


def kernel(input_token, segment_ids, token_table, type_table, pos_table,
           gamma, beta):
  tok_idx = input_token.reshape(NW, NCHUNK, CHUNK)
  seg = segment_ids.astype(jnp.float32).reshape(NW, NCHUNK * CHUNK)
  pos0 = pos_table + type_table[0][None, :]
  posx = jnp.concatenate([pos0, pos0[:CHUNK]], axis=0)
  dvec = type_table[1] - type_table[0]
  out = _sc_call(token_table, posx, tok_idx, seg, gamma, beta, dvec)
  return out.reshape(BATCH, MAX_LEN, HIDDEN)


# speedup vs baseline: 1.1092x; 1.0027x over previous
import jax
import jax.numpy as jnp
from jax import lax
from jax.experimental import pallas as pl
from jax.experimental.pallas import tpu as pltpu
from jax.experimental.pallas import tpu_sc as plsc

VOCAB = 100000
MAX_LEN = 200
HIDDEN = 128
BATCH = 1024
N_TOK = BATCH * MAX_LEN
NW = 32
TOK_PER_W = N_TOK // NW
CHUNK = 128
NCHUNK = TOK_PER_W // CHUNK
POSX = MAX_LEN + CHUNK
TRIPLES = (NCHUNK - 2) // 3
NJ = HIDDEN // 16


def _sc_body(tok_table, posx, tok_idx, seg, gamma, beta, dvec, out,
             tok_idx_v, seg_v, posx_v, buf, gv, bv, dv,
             sbuf, s2buf, mbuf, rbuf,
             tsem, osem):
  wid = lax.axis_index("s") * 2 + lax.axis_index("c")
  w_base = wid * TOK_PER_W

  pltpu.sync_copy(gamma, gv)
  pltpu.sync_copy(beta, bv)
  pltpu.sync_copy(dvec, dv)
  pltpu.sync_copy(posx, posx_v)
  pltpu.sync_copy(tok_idx.at[wid], tok_idx_v)
  pltpu.sync_copy(seg.at[wid], seg_v)
  gvs = [gv[pl.ds(16 * j, 16)] for j in range(NJ)]
  bvs = [bv[pl.ds(16 * j, 16)] for j in range(NJ)]
  dvs = [dv[pl.ds(16 * j, 16)] for j in range(NJ)]

  lanes = lax.iota(jnp.int32, 16)
  zeros16i = jnp.zeros((16,), jnp.int32)

  def issue_tok(g, s):
    pltpu.async_copy(tok_table.at[tok_idx_v.at[g]], buf.at[s], tsem.at[s])

  def wait_tok(g, s):
    pltpu.make_async_copy(tok_table.at[tok_idx_v.at[g]], buf.at[s],
                          tsem.at[s]).wait()

  def out_copy(g, s):
    base = w_base + g * CHUNK
    return pltpu.make_async_copy(buf.at[s], out.at[pl.ds(base, CHUNK)],
                                 osem.at[s])

  def compute(g, s, lbase):
    def group(grp, carry):
      @plsc.parallel_loop(0, 16, 1, unroll=8)
      def _(ti):
        t = grp * 16 + ti
        sseg = plsc.load_gather(seg_v, (zeros16i + (g * CHUNK + t),))
        prow = lbase + t
        y = [buf[s, t, pl.ds(16 * j, 16)] + posx_v[prow, pl.ds(16 * j, 16)]
             + sseg * dvs[j] for j in range(NJ)]
        for j in range(NJ):
          buf[s, t, pl.ds(16 * j, 16)] = y[j]
        tot = ((y[0] + y[1]) + (y[2] + y[3])) + ((y[4] + y[5]) + (y[6] + y[7]))
        q = [yj * yj for yj in y]
        sq = ((q[0] + q[1]) + (q[2] + q[3])) + ((q[4] + q[5]) + (q[6] + q[7]))
        sbuf[pl.ds(ti * 17, 16)] = tot
        s2buf[pl.ds(ti * 17, 16)] = sq

      zero = jnp.zeros((16,), jnp.float32)

      @plsc.parallel_loop(0, 8, 1, unroll=8,
                          carry=(lanes * 17, zero, zero, zero, zero))
      def red(k, c):
        ck, a1, a2, b1, b2 = c
        a1 = a1 + plsc.load_gather(sbuf, (ck,))
        a2 = a2 + plsc.load_gather(s2buf, (ck,))
        b1 = b1 + plsc.load_gather(sbuf, (ck + 8,))
        b2 = b2 + plsc.load_gather(s2buf, (ck + 8,))
        return ck + 1, a1, a2, b1, b2

      _, a1, a2, b1, b2 = red
      acc = a1 + b1
      acc2 = a2 + b2
      mean = acc * (1.0 / HIDDEN)
      var = acc2 * (1.0 / HIDDEN) - mean * mean
      a = var + 1e-5
      yi = jnp.int32(0x5F3759DF) - (plsc.bitcast(a, jnp.int32) >> 1)
      r = plsc.bitcast(yi, jnp.float32)
      h = a * 0.5
      for _ in range(3):
        r = r * (1.5 - h * r * r)
      mbuf[:] = mean * r
      rbuf[:] = r

      @plsc.parallel_loop(0, 16, 1, unroll=8)
      def _(ti):
        t = grp * 16 + ti
        tsplat = jnp.full((16,), ti, jnp.int32)
        mr = plsc.load_gather(mbuf, (tsplat,))
        rs = plsc.load_gather(rbuf, (tsplat,))
        for j in range(NJ):
          yj = buf[s, t, pl.ds(16 * j, 16)]
          buf[s, t, pl.ds(16 * j, 16)] = (yj * rs - mr) * gvs[j] + bvs[j]
      return carry

    lax.fori_loop(0, CHUNK // 16, group, 0)

  def wrap(x):
    return jnp.where(x >= MAX_LEN, x - MAX_LEN, x)

  def step(g, s, lbase, first):
    s2 = (s + 2) % 3
    wait_tok(g, s)
    compute(g, s, lbase)
    out_copy(g, s).start()
    if not first:
      out_copy(g - 1, s2).wait()
    issue_tok(g + 2, s2)

  issue_tok(0, 0)
  issue_tok(1, 1)

  def triple(p, lbase, first):
    g = 3 * p
    step(g, 0, lbase, first)
    lbase = wrap(lbase + CHUNK)
    step(g + 1, 1, lbase, False)
    lbase = wrap(lbase + CHUNK)
    step(g + 2, 2, lbase, False)
    return wrap(lbase + CHUNK)

  lbase = triple(0, 0, True)
  lbase = lax.fori_loop(1, TRIPLES, lambda p, lb: triple(p, lb, False), lbase)

  g = NCHUNK - 2
  wait_tok(g, 0)
  compute(g, 0, lbase)
  out_copy(g, 0).start()

  g = NCHUNK - 1
  lbase = wrap(lbase + CHUNK)
  wait_tok(g, 1)
  compute(g, 1, lbase)
  out_copy(g, 1).start()

  out_copy(NCHUNK - 3, 2).wait()
  out_copy(NCHUNK - 2, 0).wait()
  out_copy(NCHUNK - 1, 1).wait()


_sc_call = pl.kernel(
    _sc_body,
    out_type=jax.ShapeDtypeStruct((N_TOK, HIDDEN), jnp.float32),
    mesh=plsc.VectorSubcoreMesh(core_axis_name="c", subcore_axis_name="s"),
    compiler_params=pltpu.CompilerParams(needs_layout_passes=False),
    scratch_types=[
        pltpu.VMEM((NCHUNK, CHUNK), jnp.int32),
        pltpu.VMEM((NCHUNK * CHUNK,), jnp.float32),
        pltpu.VMEM((POSX, HIDDEN), jnp.float32),
        pltpu.VMEM((3, CHUNK, HIDDEN), jnp.float32),
        pltpu.VMEM((HIDDEN,), jnp.float32),
        pltpu.VMEM((HIDDEN,), jnp.float32),
        pltpu.VMEM((HIDDEN,), jnp.float32),
        pltpu.VMEM((16 * 17,), jnp.float32),
        pltpu.VMEM((16 * 17,), jnp.float32),
        pltpu.VMEM((16,), jnp.float32),
        pltpu.VMEM((16,), jnp.float32),
        pltpu.SemaphoreType.DMA((3,)),
        pltpu.SemaphoreType.DMA((3,)),
    ],
)


def kernel(input_token, segment_ids, token_table, type_table, pos_table,
           gamma, beta):
  tok_idx = input_token.reshape(NW, NCHUNK, CHUNK)
  seg = segment_ids.astype(jnp.float32).reshape(NW, NCHUNK * CHUNK)
  pos0 = pos_table + type_table[0][None, :]
  posx = jnp.concatenate([pos0, pos0[:CHUNK]], axis=0)
  dvec = type_table[1] - type_table[0]
  out = _sc_call(token_table, posx, tok_idx, seg, gamma, beta, dvec)
  return out.reshape(BATCH, MAX_LEN, HIDDEN)


# speedup vs baseline: 1.1129x; 1.0034x over previous
import jax
import jax.numpy as jnp
from jax import lax
from jax.experimental import pallas as pl
from jax.experimental.pallas import tpu as pltpu
from jax.experimental.pallas import tpu_sc as plsc

VOCAB = 100000
MAX_LEN = 200
HIDDEN = 128
BATCH = 1024
N_TOK = BATCH * MAX_LEN
NW = 32
TOK_PER_W = N_TOK // NW
CHUNK = 128
NCHUNK = TOK_PER_W // CHUNK
POSX = MAX_LEN + CHUNK
TRIPLES = (NCHUNK - 2) // 3
NJ = HIDDEN // 16


def _sc_body(tok_table, posx, tok_idx, seg, gamma, beta, dvec, out,
             tok_idx_v, seg_v, posx_v, buf, gv, bv, dv,
             sbuf, s2buf, mbuf, rbuf,
             tsem, osem):
  wid = lax.axis_index("s") * 2 + lax.axis_index("c")
  w_base = wid * TOK_PER_W

  pltpu.sync_copy(gamma, gv)
  pltpu.sync_copy(beta, bv)
  pltpu.sync_copy(dvec, dv)
  pltpu.sync_copy(posx, posx_v)
  pltpu.sync_copy(tok_idx.at[wid], tok_idx_v)
  pltpu.sync_copy(seg.at[wid], seg_v)
  gvs = [gv[pl.ds(16 * j, 16)] for j in range(NJ)]
  bvs = [bv[pl.ds(16 * j, 16)] for j in range(NJ)]
  dvs = [dv[pl.ds(16 * j, 16)] for j in range(NJ)]

  lanes = lax.iota(jnp.int32, 16)
  zeros16i = jnp.zeros((16,), jnp.int32)

  def issue_tok(g, s):
    pltpu.async_copy(tok_table.at[tok_idx_v.at[g]], buf.at[s], tsem.at[s])

  def wait_tok(g, s):
    pltpu.make_async_copy(tok_table.at[tok_idx_v.at[g]], buf.at[s],
                          tsem.at[s]).wait()

  def out_copy(g, s):
    base = w_base + g * CHUNK
    return pltpu.make_async_copy(buf.at[s], out.at[pl.ds(base, CHUNK)],
                                 osem.at[s])

  def compute(g, s, lbase):
    def group(grp, carry):
      @plsc.parallel_loop(0, 16, 1, unroll=16)
      def _(ti):
        t = grp * 16 + ti
        sseg = plsc.load_gather(seg_v, (zeros16i + (g * CHUNK + t),))
        prow = lbase + t
        y = [buf[s, t, pl.ds(16 * j, 16)] + posx_v[prow, pl.ds(16 * j, 16)]
             + sseg * dvs[j] for j in range(NJ)]
        for j in range(NJ):
          buf[s, t, pl.ds(16 * j, 16)] = y[j]
        tot = ((y[0] + y[1]) + (y[2] + y[3])) + ((y[4] + y[5]) + (y[6] + y[7]))
        q = [yj * yj for yj in y]
        sq = ((q[0] + q[1]) + (q[2] + q[3])) + ((q[4] + q[5]) + (q[6] + q[7]))
        sbuf[pl.ds(ti * 17, 16)] = tot
        s2buf[pl.ds(ti * 17, 16)] = sq

      zero = jnp.zeros((16,), jnp.float32)

      @plsc.parallel_loop(0, 8, 1, unroll=8,
                          carry=(lanes * 17, zero, zero, zero, zero))
      def red(k, c):
        ck, a1, a2, b1, b2 = c
        a1 = a1 + plsc.load_gather(sbuf, (ck,))
        a2 = a2 + plsc.load_gather(s2buf, (ck,))
        b1 = b1 + plsc.load_gather(sbuf, (ck + 8,))
        b2 = b2 + plsc.load_gather(s2buf, (ck + 8,))
        return ck + 1, a1, a2, b1, b2

      _, a1, a2, b1, b2 = red
      acc = a1 + b1
      acc2 = a2 + b2
      mean = acc * (1.0 / HIDDEN)
      var = acc2 * (1.0 / HIDDEN) - mean * mean
      a = var + 1e-5
      yi = jnp.int32(0x5F3759DF) - (plsc.bitcast(a, jnp.int32) >> 1)
      r = plsc.bitcast(yi, jnp.float32)
      h = a * 0.5
      for _ in range(3):
        r = r * (1.5 - h * r * r)
      mbuf[:] = mean * r
      rbuf[:] = r

      @plsc.parallel_loop(0, 16, 1, unroll=16)
      def _(ti):
        t = grp * 16 + ti
        tsplat = jnp.full((16,), ti, jnp.int32)
        mr = plsc.load_gather(mbuf, (tsplat,))
        rs = plsc.load_gather(rbuf, (tsplat,))
        for j in range(NJ):
          yj = buf[s, t, pl.ds(16 * j, 16)]
          buf[s, t, pl.ds(16 * j, 16)] = (yj * rs - mr) * gvs[j] + bvs[j]
      return carry

    lax.fori_loop(0, CHUNK // 16, group, 0)

  def wrap(x):
    return jnp.where(x >= MAX_LEN, x - MAX_LEN, x)

  def step(g, s, lbase, first):
    s2 = (s + 2) % 3
    wait_tok(g, s)
    compute(g, s, lbase)
    out_copy(g, s).start()
    if not first:
      out_copy(g - 1, s2).wait()
    issue_tok(g + 2, s2)

  issue_tok(0, 0)
  issue_tok(1, 1)

  def triple(p, lbase, first):
    g = 3 * p
    step(g, 0, lbase, first)
    lbase = wrap(lbase + CHUNK)
    step(g + 1, 1, lbase, False)
    lbase = wrap(lbase + CHUNK)
    step(g + 2, 2, lbase, False)
    return wrap(lbase + CHUNK)

  lbase = triple(0, 0, True)
  lbase = lax.fori_loop(1, TRIPLES, lambda p, lb: triple(p, lb, False), lbase)

  g = NCHUNK - 2
  wait_tok(g, 0)
  compute(g, 0, lbase)
  out_copy(g, 0).start()

  g = NCHUNK - 1
  lbase = wrap(lbase + CHUNK)
  wait_tok(g, 1)
  compute(g, 1, lbase)
  out_copy(g, 1).start()

  out_copy(NCHUNK - 3, 2).wait()
  out_copy(NCHUNK - 2, 0).wait()
  out_copy(NCHUNK - 1, 1).wait()


_sc_call = pl.kernel(
    _sc_body,
    out_type=jax.ShapeDtypeStruct((N_TOK, HIDDEN), jnp.float32),
    mesh=plsc.VectorSubcoreMesh(core_axis_name="c", subcore_axis_name="s"),
    compiler_params=pltpu.CompilerParams(needs_layout_passes=False),
    scratch_types=[
        pltpu.VMEM((NCHUNK, CHUNK), jnp.int32),
        pltpu.VMEM((NCHUNK * CHUNK,), jnp.float32),
        pltpu.VMEM((POSX, HIDDEN), jnp.float32),
        pltpu.VMEM((3, CHUNK, HIDDEN), jnp.float32),
        pltpu.VMEM((HIDDEN,), jnp.float32),
        pltpu.VMEM((HIDDEN,), jnp.float32),
        pltpu.VMEM((HIDDEN,), jnp.float32),
        pltpu.VMEM((16 * 17,), jnp.float32),
        pltpu.VMEM((16 * 17,), jnp.float32),
        pltpu.VMEM((16,), jnp.float32),
        pltpu.VMEM((16,), jnp.float32),
        pltpu.SemaphoreType.DMA((3,)),
        pltpu.SemaphoreType.DMA((3,)),
    ],
)


def kernel(input_token, segment_ids, token_table, type_table, pos_table,
           gamma, beta):
  tok_idx = input_token.reshape(NW, NCHUNK, CHUNK)
  seg = segment_ids.astype(jnp.float32).reshape(NW, NCHUNK * CHUNK)
  pos0 = pos_table + type_table[0][None, :]
  posx = jnp.concatenate([pos0, pos0[:CHUNK]], axis=0)
  dvec = type_table[1] - type_table[0]
  out = _sc_call(token_table, posx, tok_idx, seg, gamma, beta, dvec)
  return out.reshape(BATCH, MAX_LEN, HIDDEN)
